# baseline (speedup 1.0000x reference)
import functools

import jax
import jax.numpy as jnp
from jax import lax
from jax.experimental import pallas as pl
from jax.experimental.pallas import tpu as pltpu
from jax.experimental.pallas import tpu_sc as plsc

NC = 2
NS = 16
NW = NC * NS
BV = 1024
CH = 512
UNROLL = 2


def _make_sc_segment_sum(Vp, Ep):
    Q = Vp // 4
    assert Vp % (4 * BV) == 0
    assert Ep % (8 * CH) == 0
    n_half = (Ep // 2) // CH
    n_eighth = (Ep // 8) // CH
    groups = CH // (16 * UNROLL)
    mesh = plsc.VectorSubcoreMesh(core_axis_name="c", subcore_axis_name="s",
                                  num_cores=NC, num_subcores=NS)

    @functools.partial(
        pl.kernel,
        out_type=(jax.ShapeDtypeStruct((3, 2, Vp), jnp.float32),
                  jax.ShapeDtypeStruct((8, Vp), jnp.float32)),
        mesh=mesh,
        scratch_types=dict(
            buf1=pltpu.VMEM((Vp,), jnp.float32),
            acc_q=pltpu.VMEM((Q,), jnp.float32),
            src_b=pltpu.VMEM((CH,), jnp.int32),
            dst_b=pltpu.VMEM((CH,), jnp.int32),
        ),
        compiler_params=pltpu.CompilerParams(use_tc_tiling_on_sc=False,
                                             needs_layout_passes=False),
    )
    def seg_sum(vertsT_hbm, src_hbm, dst_hbm, zeros_hbm,
                coords_out, degs_out, buf1, acc_q, src_b, dst_b):
        cid = lax.axis_index("c")
        sid = lax.axis_index("s")
        wid = cid * NS + sid

        @pl.when(wid < 24)
        def _coord_tile():
            k = wid // 8
            r = wid % 8
            q = r // 2
            t = r % 2
            lo = q * Q
            pltpu.sync_copy(vertsT_hbm.at[k], buf1)
            pltpu.sync_copy(zeros_hbm.at[pl.ds(0, Q)], acc_q)

            def chunk(i, carry):
                base = t * (Ep // 2) + i * CH
                pltpu.sync_copy(src_hbm.at[pl.ds(base, CH)], src_b)
                pltpu.sync_copy(dst_hbm.at[pl.ds(base, CH)], dst_b)

                def group(g, c2):
                    for u in range(UNROLL):
                        off = (g * UNROLL + u) * 16
                        s = src_b[pl.ds(off, 16)]
                        d = dst_b[pl.ds(off, 16)]
                        xs = plsc.load_gather(buf1, [s])
                        xd = plsc.load_gather(buf1, [d])
                        for cc, xn in ((s, xd), (d, xs)):
                            ci = cc - lo
                            m = ci.astype(jnp.uint32) < jnp.uint32(Q)
                            ci = jnp.clip(ci, 0, Q - 1)
                            plsc.addupdate_scatter(acc_q, [ci], xn, mask=m)
                    return c2

                lax.fori_loop(0, groups, group, 0)
                return carry

            lax.fori_loop(0, n_half, chunk, 0)
            pltpu.sync_copy(acc_q, coords_out.at[k, t, pl.ds(lo, Q)])

        @pl.when(wid >= 24)
        def _deg_tile():
            d8 = wid - 24
            pltpu.sync_copy(zeros_hbm, buf1)
            ones = jnp.full((16,), 1.0, dtype=jnp.float32)

            def chunk(i, carry):
                base = d8 * (Ep // 8) + i * CH
                pltpu.sync_copy(src_hbm.at[pl.ds(base, CH)], src_b)
                pltpu.sync_copy(dst_hbm.at[pl.ds(base, CH)], dst_b)

                def group(g, c2):
                    for u in range(UNROLL):
                        off = (g * UNROLL + u) * 16
                        s = src_b[pl.ds(off, 16)]
                        d = dst_b[pl.ds(off, 16)]
                        plsc.addupdate_scatter(buf1, [s], ones)
                        plsc.addupdate_scatter(buf1, [d], ones)
                    return c2

                lax.fori_loop(0, groups, group, 0)
                return carry

            lax.fori_loop(0, n_eighth, chunk, 0)
            pltpu.sync_copy(buf1, degs_out.at[d8])

    return seg_sum


def _make_tc_body(V):
    def body(cb_ref, db_ref, vt_ref, lt_ref, wt_ref, b_ref, out_ref, mx_ref):
        phase = pl.program_id(0)
        i = pl.program_id(1)

        @pl.when(jnp.logical_and(phase == 0, i == 0))
        def _():
            mx_ref[0] = 0.0

        @pl.when(phase == 0)
        def _():
            deg = jnp.sum(db_ref[...], axis=0)
            pos = i * BV + lax.broadcasted_iota(jnp.int32, (BV,), 0)
            deg = jnp.where(pos < V, deg, 0.0)
            mx_ref[0] = jnp.maximum(mx_ref[0], jnp.max(deg))

        @pl.when(phase == 1)
        def _():
            cb = cb_ref[...]
            nb = cb[:, 0, :] + cb[:, 1, :]
            deg = jnp.sum(db_ref[...], axis=0,
                          keepdims=True)
            agg = nb - deg * vt_ref[...]
            lt = lt_ref[...]
            rot_rows = []
            for k in range(3):
                lk = jnp.concatenate(
                    [lt[k:k + 1], lt[3 + k:4 + k], lt[6 + k:7 + k]], axis=0)
                rot_rows.append(jnp.sum(agg * lk, axis=0, keepdims=True))
            rot_t = jnp.concatenate(rot_rows, axis=0)
            out = lax.dot_general(
                rot_t, wt_ref[...], (((0,), (0,)), ((), ())),
                preferred_element_type=jnp.float32)
            out_ref[...] = out + mx_ref[0] * b_ref[...]

    return body


def _make_tc_finish(V, Vp, D_OUT):
    assert Vp % BV == 0
    nb = (V + BV - 1) // BV
    assert nb * BV <= Vp
    return pl.pallas_call(
        _make_tc_body(V),
        grid=(2, nb),
        in_specs=[
            pl.BlockSpec((3, 2, BV), lambda p, i: (0, 0, i * p)),
            pl.BlockSpec((8, BV), lambda p, i: (0, i)),
            pl.BlockSpec((3, BV), lambda p, i: (0, i * p)),
            pl.BlockSpec((9, BV), lambda p, i: (0, i * p)),
            pl.BlockSpec((3, D_OUT), lambda p, i: (0, 0)),
            pl.BlockSpec((1, D_OUT), lambda p, i: (0, 0)),
        ],
        out_specs=pl.BlockSpec((BV, D_OUT), lambda p, i: (i * p, 0)),
        out_shape=jax.ShapeDtypeStruct((V, D_OUT), jnp.float32),
        scratch_shapes=[pltpu.SMEM((1,), jnp.float32)],
    )


def kernel(verts, edges, lrf, W, b):
    V = verts.shape[0]
    E = edges.shape[0]
    D_OUT = W.shape[0]

    Vp = ((V + 4 * BV) // (4 * BV)) * (4 * BV)
    Ep = ((E + 8 * CH - 1) // (8 * CH)) * (8 * CH)

    vertsT = jnp.zeros((3, Vp), dtype=jnp.float32).at[:, :V].set(verts.T)
    src = jnp.full((Ep,), V, dtype=jnp.int32).at[:E].set(edges[:, 0])
    dst = jnp.full((Ep,), V, dtype=jnp.int32).at[:E].set(edges[:, 1])
    zeros = jnp.zeros((Vp,), dtype=jnp.float32)

    seg_sum = _make_sc_segment_sum(Vp, Ep)
    coords, degs = seg_sum(vertsT, src, dst, zeros)

    lt = jnp.zeros((9, Vp), dtype=jnp.float32).at[:, :V].set(
        lrf.reshape(V, 9).T)

    finish = _make_tc_finish(V, Vp, D_OUT)
    return finish(coords, degs, vertsT, lt, W.T, b.reshape(1, D_OUT))


# speedup vs baseline: 2.7600x; 2.7600x over previous
import functools

import jax
import jax.numpy as jnp
from jax import lax
from jax.experimental import pallas as pl
from jax.experimental.pallas import tpu as pltpu
from jax.experimental.pallas import tpu_sc as plsc

NC = 2
NS = 16
NW = NC * NS
BV = 1024
CH = 512
UNROLL = 4


def _make_sc_segment_sum(Vp, Ep):
    Q = Vp // 4
    assert Vp % (4 * BV) == 0
    assert Ep % (8 * CH) == 0
    n_half = (Ep // 2) // CH
    n_eighth = (Ep // 8) // CH
    assert n_half % 2 == 0 and n_eighth % 2 == 0
    groups = CH // (16 * UNROLL)
    mesh = plsc.VectorSubcoreMesh(core_axis_name="c", subcore_axis_name="s",
                                  num_cores=NC, num_subcores=NS)

    @functools.partial(
        pl.kernel,
        out_type=(jax.ShapeDtypeStruct((3, 2, Vp), jnp.float32),
                  jax.ShapeDtypeStruct((8, Vp), jnp.float32)),
        mesh=mesh,
        scratch_types=dict(
            buf1=pltpu.VMEM((Vp,), jnp.float32),
            acc_q=pltpu.VMEM((Q,), jnp.float32),
            src_b=pltpu.VMEM((2, CH), jnp.int32),
            dst_b=pltpu.VMEM((2, CH), jnp.int32),
            sems=pltpu.SemaphoreType.DMA((2, 2)),
        ),
        compiler_params=pltpu.CompilerParams(use_tc_tiling_on_sc=False,
                                             needs_layout_passes=False),
    )
    def seg_sum(vertsT_hbm, src_hbm, dst_hbm, zeros_hbm,
                coords_out, degs_out, buf1, acc_q, src_b, dst_b, sems):
        cid = lax.axis_index("c")
        sid = lax.axis_index("s")
        wid = cid * NS + sid

        def start(slot, base):
            pltpu.async_copy(src_hbm.at[pl.ds(base, CH)], src_b.at[slot],
                             sems.at[slot, 0])
            pltpu.async_copy(dst_hbm.at[pl.ds(base, CH)], dst_b.at[slot],
                             sems.at[slot, 1])

        def wait(slot):
            pltpu.make_async_copy(src_hbm.at[pl.ds(0, CH)], src_b.at[slot],
                                  sems.at[slot, 0]).wait()
            pltpu.make_async_copy(dst_hbm.at[pl.ds(0, CH)], dst_b.at[slot],
                                  sems.at[slot, 1]).wait()

        def run_chunks(base0, n_chunks, compute):
            start(0, base0)

            def body(i2, carry):
                c0 = 2 * i2
                start(1, base0 + (c0 + 1) * CH)
                wait(0)
                compute(0)

                @pl.when(c0 + 2 < n_chunks)
                def _():
                    start(0, base0 + (c0 + 2) * CH)

                wait(1)
                compute(1)
                return carry

            lax.fori_loop(0, n_chunks // 2, body, 0)

        @pl.when(wid < 24)
        def _coord_tile():
            k = wid // 8
            r = wid % 8
            q = r // 2
            t = r % 2
            lo = q * Q
            pltpu.sync_copy(vertsT_hbm.at[k], buf1)
            pltpu.sync_copy(zeros_hbm.at[pl.ds(0, Q)], acc_q)

            def compute(slot):
                def group(g, c2):
                    for u in range(UNROLL):
                        off = (g * UNROLL + u) * 16
                        s = src_b[slot, pl.ds(off, 16)]
                        d = dst_b[slot, pl.ds(off, 16)]
                        xs = plsc.load_gather(buf1, [s])
                        xd = plsc.load_gather(buf1, [d])
                        for cc, xn in ((s, xd), (d, xs)):
                            ci = cc - lo
                            m = ci.astype(jnp.uint32) < jnp.uint32(Q)
                            ci = jnp.clip(ci, 0, Q - 1)
                            plsc.addupdate_scatter(acc_q, [ci], xn, mask=m)
                    return c2

                lax.fori_loop(0, groups, group, 0)

            run_chunks(t * (Ep // 2), n_half, compute)
            pltpu.sync_copy(acc_q, coords_out.at[k, t, pl.ds(lo, Q)])

        @pl.when(wid >= 24)
        def _deg_tile():
            d8 = wid - 24
            pltpu.sync_copy(zeros_hbm, buf1)
            ones = jnp.full((16,), 1.0, dtype=jnp.float32)

            def compute(slot):
                def group(g, c2):
                    for u in range(UNROLL):
                        off = (g * UNROLL + u) * 16
                        s = src_b[slot, pl.ds(off, 16)]
                        d = dst_b[slot, pl.ds(off, 16)]
                        plsc.addupdate_scatter(buf1, [s], ones)
                        plsc.addupdate_scatter(buf1, [d], ones)
                    return c2

                lax.fori_loop(0, groups, group, 0)

            run_chunks(d8 * (Ep // 8), n_eighth, compute)
            pltpu.sync_copy(buf1, degs_out.at[d8])

    return seg_sum


def _make_tc_body(V):
    def body(cb_ref, db_ref, vt_ref, lt_ref, wt_ref, b_ref, out_ref, mx_ref):
        phase = pl.program_id(0)
        i = pl.program_id(1)

        @pl.when(jnp.logical_and(phase == 0, i == 0))
        def _():
            mx_ref[0] = 0.0

        @pl.when(phase == 0)
        def _():
            deg = jnp.sum(db_ref[...], axis=0)
            pos = i * BV + lax.broadcasted_iota(jnp.int32, (BV,), 0)
            deg = jnp.where(pos < V, deg, 0.0)
            mx_ref[0] = jnp.maximum(mx_ref[0], jnp.max(deg))

        @pl.when(phase == 1)
        def _():
            cb = cb_ref[...]
            nb = cb[:, 0, :] + cb[:, 1, :]
            deg = jnp.sum(db_ref[...], axis=0,
                          keepdims=True)
            agg = nb - deg * vt_ref[...]
            lt = lt_ref[...]
            rot_rows = []
            for k in range(3):
                lk = jnp.concatenate(
                    [lt[k:k + 1], lt[3 + k:4 + k], lt[6 + k:7 + k]], axis=0)
                rot_rows.append(jnp.sum(agg * lk, axis=0, keepdims=True))
            rot_t = jnp.concatenate(rot_rows, axis=0)
            out = lax.dot_general(
                rot_t, wt_ref[...], (((0,), (0,)), ((), ())),
                preferred_element_type=jnp.float32)
            out_ref[...] = out + mx_ref[0] * b_ref[...]

    return body


def _make_tc_finish(V, Vp, D_OUT):
    assert Vp % BV == 0
    nb = (V + BV - 1) // BV
    assert nb * BV <= Vp
    return pl.pallas_call(
        _make_tc_body(V),
        grid=(2, nb),
        in_specs=[
            pl.BlockSpec((3, 2, BV), lambda p, i: (0, 0, i * p)),
            pl.BlockSpec((8, BV), lambda p, i: (0, i)),
            pl.BlockSpec((3, BV), lambda p, i: (0, i * p)),
            pl.BlockSpec((9, BV), lambda p, i: (0, i * p)),
            pl.BlockSpec((3, D_OUT), lambda p, i: (0, 0)),
            pl.BlockSpec((1, D_OUT), lambda p, i: (0, 0)),
        ],
        out_specs=pl.BlockSpec((BV, D_OUT), lambda p, i: (i * p, 0)),
        out_shape=jax.ShapeDtypeStruct((V, D_OUT), jnp.float32),
        scratch_shapes=[pltpu.SMEM((1,), jnp.float32)],
    )


def kernel(verts, edges, lrf, W, b):
    V = verts.shape[0]
    E = edges.shape[0]
    D_OUT = W.shape[0]

    Vp = ((V + 4 * BV) // (4 * BV)) * (4 * BV)
    Ep = ((E + 8 * CH - 1) // (8 * CH)) * (8 * CH)

    vertsT = jnp.zeros((3, Vp), dtype=jnp.float32).at[:, :V].set(verts.T)
    src = jnp.full((Ep,), V, dtype=jnp.int32).at[:E].set(edges[:, 0])
    dst = jnp.full((Ep,), V, dtype=jnp.int32).at[:E].set(edges[:, 1])
    zeros = jnp.zeros((Vp,), dtype=jnp.float32)

    seg_sum = _make_sc_segment_sum(Vp, Ep)
    coords, degs = seg_sum(vertsT, src, dst, zeros)

    lt = jnp.zeros((9, Vp), dtype=jnp.float32).at[:, :V].set(
        lrf.reshape(V, 9).T)

    finish = _make_tc_finish(V, Vp, D_OUT)
    return finish(coords, degs, vertsT, lt, W.T, b.reshape(1, D_OUT))


# speedup vs baseline: 2.8199x; 1.0217x over previous
import functools

import jax
import jax.numpy as jnp
from jax import lax
from jax.experimental import pallas as pl
from jax.experimental.pallas import tpu as pltpu
from jax.experimental.pallas import tpu_sc as plsc

NC = 2
NS = 16
NW = NC * NS
BV = 1024
CH = 512
UNROLL = 8


def _make_sc_segment_sum(Vp, Ep):
    Q = Vp // 4
    assert Vp % (4 * BV) == 0
    assert Ep % (8 * CH) == 0
    n_half = (Ep // 2) // CH
    n_eighth = (Ep // 8) // CH
    assert n_half % 2 == 0 and n_eighth % 2 == 0
    groups = CH // (16 * UNROLL)
    mesh = plsc.VectorSubcoreMesh(core_axis_name="c", subcore_axis_name="s",
                                  num_cores=NC, num_subcores=NS)

    @functools.partial(
        pl.kernel,
        out_type=(jax.ShapeDtypeStruct((3, 2, Vp), jnp.float32),
                  jax.ShapeDtypeStruct((8, Vp), jnp.float32)),
        mesh=mesh,
        scratch_types=dict(
            buf1=pltpu.VMEM((Vp,), jnp.float32),
            acc_q=pltpu.VMEM((Q,), jnp.float32),
            src_b=pltpu.VMEM((2, CH), jnp.int32),
            dst_b=pltpu.VMEM((2, CH), jnp.int32),
            sems=pltpu.SemaphoreType.DMA((2, 2)),
        ),
        compiler_params=pltpu.CompilerParams(use_tc_tiling_on_sc=False,
                                             needs_layout_passes=False),
    )
    def seg_sum(vertsT_hbm, src_hbm, dst_hbm, zeros_hbm,
                coords_out, degs_out, buf1, acc_q, src_b, dst_b, sems):
        cid = lax.axis_index("c")
        sid = lax.axis_index("s")
        wid = cid * NS + sid

        def start(slot, base):
            pltpu.async_copy(src_hbm.at[pl.ds(base, CH)], src_b.at[slot],
                             sems.at[slot, 0])
            pltpu.async_copy(dst_hbm.at[pl.ds(base, CH)], dst_b.at[slot],
                             sems.at[slot, 1])

        def wait(slot):
            pltpu.make_async_copy(src_hbm.at[pl.ds(0, CH)], src_b.at[slot],
                                  sems.at[slot, 0]).wait()
            pltpu.make_async_copy(dst_hbm.at[pl.ds(0, CH)], dst_b.at[slot],
                                  sems.at[slot, 1]).wait()

        def run_chunks(base0, n_chunks, compute):
            start(0, base0)

            def body(i2, carry):
                c0 = 2 * i2
                start(1, base0 + (c0 + 1) * CH)
                wait(0)
                compute(0)

                @pl.when(c0 + 2 < n_chunks)
                def _():
                    start(0, base0 + (c0 + 2) * CH)

                wait(1)
                compute(1)
                return carry

            lax.fori_loop(0, n_chunks // 2, body, 0)

        @pl.when(wid < 24)
        def _coord_tile():
            k = wid // 8
            r = wid % 8
            q = r // 2
            t = r % 2
            lo = q * Q
            pltpu.sync_copy(vertsT_hbm.at[k], buf1)
            pltpu.sync_copy(zeros_hbm.at[pl.ds(0, Q)], acc_q)

            def compute(slot):
                def group(g, c2):
                    for u in range(UNROLL):
                        off = (g * UNROLL + u) * 16
                        s = src_b[slot, pl.ds(off, 16)]
                        d = dst_b[slot, pl.ds(off, 16)]
                        xs = plsc.load_gather(buf1, [s])
                        xd = plsc.load_gather(buf1, [d])
                        for cc, xn in ((s, xd), (d, xs)):
                            ci = cc - lo
                            m = ci.astype(jnp.uint32) < jnp.uint32(Q)
                            ci = jnp.where(m, ci, 0)
                            plsc.addupdate_scatter(acc_q, [ci], xn, mask=m)
                    return c2

                lax.fori_loop(0, groups, group, 0)

            run_chunks(t * (Ep // 2), n_half, compute)
            pltpu.sync_copy(acc_q, coords_out.at[k, t, pl.ds(lo, Q)])

        @pl.when(wid >= 24)
        def _deg_tile():
            d8 = wid - 24
            pltpu.sync_copy(zeros_hbm, buf1)
            ones = jnp.full((16,), 1.0, dtype=jnp.float32)

            def compute(slot):
                def group(g, c2):
                    for u in range(UNROLL):
                        off = (g * UNROLL + u) * 16
                        s = src_b[slot, pl.ds(off, 16)]
                        d = dst_b[slot, pl.ds(off, 16)]
                        plsc.addupdate_scatter(buf1, [s], ones)
                        plsc.addupdate_scatter(buf1, [d], ones)
                    return c2

                lax.fori_loop(0, groups, group, 0)

            run_chunks(d8 * (Ep // 8), n_eighth, compute)
            pltpu.sync_copy(buf1, degs_out.at[d8])

    return seg_sum


def _make_tc_body(V):
    def body(cb_ref, db_ref, vt_ref, lt_ref, wt_ref, b_ref, out_ref, mx_ref):
        phase = pl.program_id(0)
        i = pl.program_id(1)

        @pl.when(jnp.logical_and(phase == 0, i == 0))
        def _():
            mx_ref[0] = 0.0

        @pl.when(phase == 0)
        def _():
            deg = jnp.sum(db_ref[...], axis=0)
            pos = i * BV + lax.broadcasted_iota(jnp.int32, (BV,), 0)
            deg = jnp.where(pos < V, deg, 0.0)
            mx_ref[0] = jnp.maximum(mx_ref[0], jnp.max(deg))

        @pl.when(phase == 1)
        def _():
            cb = cb_ref[...]
            nb = cb[:, 0, :] + cb[:, 1, :]
            deg = jnp.sum(db_ref[...], axis=0,
                          keepdims=True)
            agg = nb - deg * vt_ref[...]
            lt = lt_ref[...]
            rot_rows = []
            for k in range(3):
                lk = jnp.concatenate(
                    [lt[k:k + 1], lt[3 + k:4 + k], lt[6 + k:7 + k]], axis=0)
                rot_rows.append(jnp.sum(agg * lk, axis=0, keepdims=True))
            rot_t = jnp.concatenate(rot_rows, axis=0)
            out = lax.dot_general(
                rot_t, wt_ref[...], (((0,), (0,)), ((), ())),
                preferred_element_type=jnp.float32)
            out_ref[...] = out + mx_ref[0] * b_ref[...]

    return body


def _make_tc_finish(V, Vp, D_OUT):
    assert Vp % BV == 0
    nb = (V + BV - 1) // BV
    assert nb * BV <= Vp
    return pl.pallas_call(
        _make_tc_body(V),
        grid=(2, nb),
        in_specs=[
            pl.BlockSpec((3, 2, BV), lambda p, i: (0, 0, i * p)),
            pl.BlockSpec((8, BV), lambda p, i: (0, i)),
            pl.BlockSpec((3, BV), lambda p, i: (0, i * p)),
            pl.BlockSpec((9, BV), lambda p, i: (0, i * p)),
            pl.BlockSpec((3, D_OUT), lambda p, i: (0, 0)),
            pl.BlockSpec((1, D_OUT), lambda p, i: (0, 0)),
        ],
        out_specs=pl.BlockSpec((BV, D_OUT), lambda p, i: (i * p, 0)),
        out_shape=jax.ShapeDtypeStruct((V, D_OUT), jnp.float32),
        scratch_shapes=[pltpu.SMEM((1,), jnp.float32)],
    )


def kernel(verts, edges, lrf, W, b):
    V = verts.shape[0]
    E = edges.shape[0]
    D_OUT = W.shape[0]

    Vp = ((V + 4 * BV) // (4 * BV)) * (4 * BV)
    Ep = ((E + 8 * CH - 1) // (8 * CH)) * (8 * CH)

    vertsT = jnp.zeros((3, Vp), dtype=jnp.float32).at[:, :V].set(verts.T)
    src = jnp.full((Ep,), V, dtype=jnp.int32).at[:E].set(edges[:, 0])
    dst = jnp.full((Ep,), V, dtype=jnp.int32).at[:E].set(edges[:, 1])
    zeros = jnp.zeros((Vp,), dtype=jnp.float32)

    seg_sum = _make_sc_segment_sum(Vp, Ep)
    coords, degs = seg_sum(vertsT, src, dst, zeros)

    lt = jnp.zeros((9, Vp), dtype=jnp.float32).at[:, :V].set(
        lrf.reshape(V, 9).T)

    finish = _make_tc_finish(V, Vp, D_OUT)
    return finish(coords, degs, vertsT, lt, W.T, b.reshape(1, D_OUT))
